# baseline (speedup 1.0000x reference)
import functools

import jax
import jax.numpy as jnp
from jax import lax
from jax.experimental import pallas as pl
from jax.experimental.pallas import tpu as pltpu
from jax.experimental.pallas import tpu_sc as plsc

D_VOCAB = 100000
D_MODEL = 1024
B_ROWS = 4
B_COLS = 4096

NC = 2
NS = 16
NW = NC * NS
B_PER_W = B_ROWS * B_COLS // NW
W_PER_ROW = B_COLS // B_PER_W

K = 32
NBUF = 3
GA = 1
N_CHUNKS = B_PER_W // K


def _make_embed_kernel():
  mesh = plsc.VectorSubcoreMesh(
      core_axis_name="c", subcore_axis_name="s", num_cores=NC)

  scratch = [pltpu.VMEM((B_PER_W,), jnp.int32)]
  scratch += [pltpu.VMEM((K, D_MODEL), jnp.float32) for _ in range(NBUF)]
  scratch += [pltpu.SemaphoreType.DMA for _ in range(2 * NBUF)]

  @functools.partial(
      pl.kernel,
      mesh=mesh,
      out_type=jax.ShapeDtypeStruct((B_ROWS, B_COLS, D_MODEL), jnp.float32),
      scratch_types=scratch,
  )
  def embed(table_hbm, tok_hbm, out_hbm, idx_v, *bufs_and_sems):
    bufs = bufs_and_sems[:NBUF]
    g_sems = bufs_and_sems[NBUF:2 * NBUF]
    p_sems = bufs_and_sems[2 * NBUF:]

    wid = lax.axis_index("s") * NC + lax.axis_index("c")
    brow = wid // W_PER_ROW
    bcol = (wid % W_PER_ROW) * B_PER_W

    pltpu.sync_copy(tok_hbm.at[brow, pl.ds(bcol, B_PER_W)], idx_v)

    def gather_dma(c, slot):
      return pltpu.make_async_copy(
          table_hbm.at[idx_v.at[pl.ds(c * K, K)]], bufs[slot], g_sems[slot])

    def put_dma(c, slot):
      return pltpu.make_async_copy(
          bufs[slot], out_hbm.at[brow, pl.ds(bcol + c * K, K)], p_sems[slot])

    for j in range(min(GA, N_CHUNKS)):
      gather_dma(j, j % NBUF).start()
    for c in range(N_CHUNKS):
      j = c + GA
      if j < N_CHUNKS:
        if j >= NBUF:
          put_dma(j - NBUF, j % NBUF).wait()
        gather_dma(j, j % NBUF).start()
      gather_dma(c, c % NBUF).wait()
      put_dma(c, c % NBUF).start()
    for c in range(max(N_CHUNKS - NBUF, 0), N_CHUNKS):
      put_dma(c, c % NBUF).wait()

  return embed


_embed = _make_embed_kernel()


def kernel(tokens, W_E):
  return _embed(W_E, tokens)


# speedup vs baseline: 1.0159x; 1.0159x over previous
import functools

import jax
import jax.numpy as jnp
from jax import lax
from jax.experimental import pallas as pl
from jax.experimental.pallas import tpu as pltpu
from jax.experimental.pallas import tpu_sc as plsc

D_VOCAB = 100000
D_MODEL = 1024
B_TOTAL = 4 * 4096

NC = 2
NS = 16
NW = NC * NS
B_PER_W = B_TOTAL // NW

K = 32
NBUF = 3
N_CHUNKS = B_PER_W // K


def _make_embed_kernel():
  mesh = plsc.VectorSubcoreMesh(
      core_axis_name="c", subcore_axis_name="s", num_cores=NC)

  scratch = [pltpu.VMEM((B_PER_W,), jnp.int32)]
  scratch += [pltpu.VMEM((K, D_MODEL), jnp.float32) for _ in range(NBUF)]
  scratch += [pltpu.SemaphoreType.DMA for _ in range(2 * NBUF)]

  @functools.partial(
      pl.kernel,
      mesh=mesh,
      out_type=jax.ShapeDtypeStruct((B_TOTAL, D_MODEL), jnp.float32),
      scratch_types=scratch,
  )
  def embed(table_hbm, idx_hbm, out_hbm, idx_v, *bufs_and_sems):
    bufs = bufs_and_sems[:NBUF]
    g_sems = bufs_and_sems[NBUF:2 * NBUF]
    p_sems = bufs_and_sems[2 * NBUF:]

    wid = lax.axis_index("s") * NC + lax.axis_index("c")
    base = wid * B_PER_W

    pltpu.sync_copy(idx_hbm.at[pl.ds(base, B_PER_W)], idx_v)

    def gather_dma(c, slot):
      return pltpu.make_async_copy(
          table_hbm.at[idx_v.at[pl.ds(c * K, K)]], bufs[slot], g_sems[slot])

    def put_dma(c, slot):
      return pltpu.make_async_copy(
          bufs[slot], out_hbm.at[pl.ds(base + c * K, K)], p_sems[slot])

    for b in range(min(NBUF, N_CHUNKS)):
      gather_dma(b, b).start()
    for c in range(N_CHUNKS):
      slot = c % NBUF
      gather_dma(c, slot).wait()
      put_dma(c, slot).start()
      nxt = c + NBUF
      if nxt < N_CHUNKS:
        put_dma(c, slot).wait()
        gather_dma(nxt, slot).start()
    for c in range(max(N_CHUNKS - NBUF, 0), N_CHUNKS):
      put_dma(c, c % NBUF).wait()

  return embed


_embed = _make_embed_kernel()


@jax.jit
def kernel(tokens, W_E):
  idx = tokens.reshape(-1).astype(jnp.int32)
  out = _embed(W_E, idx)
  return out.reshape(tokens.shape[0], tokens.shape[1], D_MODEL)
